# baseline (speedup 1.0000x reference)
import functools

import jax
import jax.numpy as jnp
from jax import lax
from jax.experimental import pallas as pl
from jax.experimental.pallas import tpu as pltpu
from jax.experimental.pallas import tpu_sc as plsc

CHUNK = 128


def _build(V, D, B):
  info = plsc.get_sparse_core_info()
  NW = info.num_cores * info.num_subcores
  b_per_w = B // NW
  n_chunks = b_per_w // CHUNK

  mesh = plsc.VectorSubcoreMesh(core_axis_name="c", subcore_axis_name="s")

  @functools.partial(
      pl.kernel,
      mesh=mesh,
      out_type=jax.ShapeDtypeStruct((B, D), jnp.float32),
      compiler_params=pltpu.CompilerParams(use_tc_tiling_on_sc=False),
      scratch_types=[
          pltpu.VMEM((n_chunks, CHUNK), jnp.int32),
          pltpu.VMEM((b_per_w, D), jnp.float32),
          pltpu.SemaphoreType.DMA,
      ],
  )
  def k(table_hbm, idx_hbm, out_hbm, idx_v, rows_v, sem):
    wid = lax.axis_index("s") * info.num_cores + lax.axis_index("c")
    base = wid * b_per_w
    pltpu.sync_copy(idx_hbm.at[pl.ds(wid * n_chunks, n_chunks)], idx_v)
    copies = []
    for j in range(n_chunks):
      copies.append(
          pltpu.async_copy(
              table_hbm.at[idx_v.at[j]],
              rows_v.at[pl.ds(j * CHUNK, CHUNK)],
              sem,
          ))
    for c in copies:
      c.wait()
    pltpu.sync_copy(rows_v, out_hbm.at[pl.ds(base, b_per_w)])

  return k


def kernel(t, embed_weight):
  V, D = embed_weight.shape
  B = t.shape[0]
  idx = t.astype(jnp.int32).reshape(B // CHUNK, CHUNK)
  return _build(V, D, B)(embed_weight, idx)


# speedup vs baseline: 1.0027x; 1.0027x over previous
import functools

import jax
import jax.numpy as jnp
from jax import lax
from jax.experimental import pallas as pl
from jax.experimental.pallas import tpu as pltpu
from jax.experimental.pallas import tpu_sc as plsc

CHUNK = 128


def _build(V, D, B):
  info = plsc.get_sparse_core_info()
  NW = info.num_cores * info.num_subcores
  b_per_w = B // NW
  n_chunks = b_per_w // CHUNK

  mesh = plsc.VectorSubcoreMesh(core_axis_name="c", subcore_axis_name="s")

  @functools.partial(
      pl.kernel,
      mesh=mesh,
      out_type=jax.ShapeDtypeStruct((B, D), jnp.float32),
      compiler_params=pltpu.CompilerParams(use_tc_tiling_on_sc=False),
      scratch_types=[
          pltpu.VMEM((n_chunks, CHUNK), jnp.int32),
          pltpu.VMEM((b_per_w, D), jnp.float32),
          pltpu.SemaphoreType.DMA((n_chunks,)),
          pltpu.SemaphoreType.DMA((n_chunks,)),
      ],
  )
  def k(table_hbm, idx_hbm, out_hbm, idx_v, rows_v, gsem, wsem):
    wid = lax.axis_index("s") * info.num_cores + lax.axis_index("c")
    base = wid * b_per_w
    pltpu.sync_copy(idx_hbm.at[pl.ds(wid * n_chunks, n_chunks)], idx_v)
    gathers = []
    for j in range(n_chunks):
      gathers.append(
          pltpu.async_copy(
              table_hbm.at[idx_v.at[j]],
              rows_v.at[pl.ds(j * CHUNK, CHUNK)],
              gsem.at[j],
          ))
    writes = []
    for j in range(n_chunks):
      gathers[j].wait()
      writes.append(
          pltpu.async_copy(
              rows_v.at[pl.ds(j * CHUNK, CHUNK)],
              out_hbm.at[pl.ds(base + j * CHUNK, CHUNK)],
              wsem.at[j],
          ))
    for c in writes:
      c.wait()

  return k


def kernel(t, embed_weight):
  V, D = embed_weight.shape
  B = t.shape[0]
  idx = t.astype(jnp.int32).reshape(B // CHUNK, CHUNK)
  return _build(V, D, B)(embed_weight, idx)


# speedup vs baseline: 1.9894x; 1.9840x over previous
import functools

import jax
import jax.numpy as jnp
from jax import lax
from jax.experimental import pallas as pl
from jax.experimental.pallas import tpu as pltpu
from jax.experimental.pallas import tpu_sc as plsc

LANES = 16
OUT_CHUNK = 8192


def _build(V, D, B):
  info = plsc.get_sparse_core_info()
  NC = info.num_cores
  NW = NC * info.num_subcores
  d_per_w = D // NW
  n_chunks = B // OUT_CHUNK

  mesh = plsc.VectorSubcoreMesh(core_axis_name="c", subcore_axis_name="s")

  @functools.partial(
      pl.kernel,
      mesh=mesh,
      out_type=jax.ShapeDtypeStruct((D, B), jnp.float32),
      compiler_params=pltpu.CompilerParams(
          use_tc_tiling_on_sc=True, needs_layout_passes=False),
      scratch_types=[
          pltpu.VMEM((V,), jnp.float32),
          pltpu.VMEM((B,), jnp.int32),
          pltpu.VMEM((OUT_CHUNK,), jnp.float32),
      ],
  )
  def k(tw_hbm, t_hbm, out_hbm, row_v, idx_v, o_v):
    wid = lax.axis_index("s") * NC + lax.axis_index("c")
    pltpu.sync_copy(t_hbm, idx_v)
    for rr in range(d_per_w):
      d = wid * d_per_w + rr
      pltpu.sync_copy(tw_hbm.at[d], row_v)

      for ci in range(n_chunks):

        def body(j, carry, ci=ci):
          tvec = idx_v[pl.ds(ci * OUT_CHUNK + j * LANES, LANES)]
          o_v[pl.ds(j * LANES, LANES)] = plsc.load_gather(row_v, [tvec])
          return carry

        lax.fori_loop(0, OUT_CHUNK // LANES, body, 0)
        pltpu.sync_copy(o_v, out_hbm.at[d, pl.ds(ci * OUT_CHUNK, OUT_CHUNK)])

  return k


def kernel(t, embed_weight):
  V, D = embed_weight.shape
  B = t.shape[0]
  out_t = _build(V, D, B)(embed_weight.T, t.astype(jnp.int32))
  return out_t.T


# speedup vs baseline: 2.7287x; 1.3717x over previous
import functools

import jax
import jax.numpy as jnp
from jax import lax
from jax.experimental import pallas as pl
from jax.experimental.pallas import tpu as pltpu
from jax.experimental.pallas import tpu_sc as plsc

LANES = 16
OUT_CHUNK = 4096


def _build(V, D, B):
  info = plsc.get_sparse_core_info()
  NC = info.num_cores
  NW = NC * info.num_subcores
  d_per_w = D // NW
  n_chunks = B // OUT_CHUNK

  mesh = plsc.VectorSubcoreMesh(core_axis_name="c", subcore_axis_name="s")

  @functools.partial(
      pl.kernel,
      mesh=mesh,
      out_type=jax.ShapeDtypeStruct((D, B), jnp.float32),
      compiler_params=pltpu.CompilerParams(
          use_tc_tiling_on_sc=True, needs_layout_passes=False),
      scratch_types=[
          pltpu.VMEM((V,), jnp.float32),
          pltpu.VMEM((B,), jnp.int32),
          pltpu.VMEM((OUT_CHUNK,), jnp.float32),
          pltpu.VMEM((OUT_CHUNK,), jnp.float32),
          pltpu.SemaphoreType.DMA((2,)),
      ],
  )
  def k(tw_hbm, t_hbm, out_hbm, row_v, idx_v, o0_v, o1_v, wsem):
    wid = lax.axis_index("s") * NC + lax.axis_index("c")
    pltpu.sync_copy(t_hbm, idx_v)
    pltpu.sync_copy(tw_hbm.at[wid * d_per_w], row_v)

    o_bufs = (o0_v, o1_v)
    pending = [None, None]
    for rr in range(d_per_w):
      d = wid * d_per_w + rr
      for ci in range(n_chunks):
        slot = (rr * n_chunks + ci) % 2
        if pending[slot] is not None:
          pending[slot].wait()
        o_ref = o_bufs[slot]

        @plsc.parallel_loop(0, OUT_CHUNK, step=LANES, unroll=8)
        def _(j, ci=ci, o_ref=o_ref):
          tvec = idx_v[pl.ds(ci * OUT_CHUNK + j, LANES)]
          o_ref[pl.ds(j, LANES)] = plsc.load_gather(row_v, [tvec])

        pending[slot] = pltpu.async_copy(
            o_ref, out_hbm.at[d, pl.ds(ci * OUT_CHUNK, OUT_CHUNK)],
            wsem.at[slot])
      if rr + 1 < d_per_w:
        pltpu.sync_copy(tw_hbm.at[d + 1], row_v)
    for c in pending:
      if c is not None:
        c.wait()

  return k


def kernel(t, embed_weight):
  V, D = embed_weight.shape
  B = t.shape[0]
  out_t = _build(V, D, B)(embed_weight.T, t.astype(jnp.int32))
  return out_t.T
